# baseline (speedup 1.0000x reference)
import functools

import jax
import jax.numpy as jnp
import numpy as np
from jax import lax
from jax.experimental import pallas as pl
from jax.experimental.pallas import tpu as pltpu
from jax.experimental.pallas import tpu_sc as plsc

_EMBED_DIM = 1024
_MAX_LENGTH = 4096
_SEQ_LEN = 4096

_NUM_CORES = 2
_NUM_SUBCORES = 16
_NUM_WORKERS = _NUM_CORES * _NUM_SUBCORES
_ROWS_PER_WORKER = _SEQ_LEN // _NUM_WORKERS
_CHUNK = 16
_STEPS = _ROWS_PER_WORKER // _CHUNK
_LANES = 16
_VECS_PER_CHUNK = _CHUNK * _EMBED_DIM // _LANES


def _pe_shifted_np() -> np.ndarray:
    d_model, max_len = _EMBED_DIM, _MAX_LENGTH
    position = np.arange(max_len, dtype=np.float64)[:, None]
    div_term = np.exp(
        np.arange(0, d_model, 2, dtype=np.float64) * (-np.log(10000.0) / d_model)
    )
    pe = np.zeros((max_len, d_model), dtype=np.float64)
    pe[:, 0::2] = np.sin(position * div_term)
    pe[:, 1::2] = np.cos(position * div_term)
    out = np.zeros((_SEQ_LEN, d_model), dtype=np.float64)
    out[1:] = pe[: _SEQ_LEN - 1]
    return out.astype(np.float32)


_PE_SHIFTED = _pe_shifted_np()


def _sc_body(bin_hbm, table_hbm, pe_hbm, out_hbm, idx_v, pe_v, gat_v,
             sem_pe, sem_gat):
    wid = lax.axis_index("s") * _NUM_CORES + lax.axis_index("c")
    base = wid * _ROWS_PER_WORKER

    for t in range(_STEPS):
        r0 = base + t * _CHUNK
        pltpu.sync_copy(bin_hbm.at[pl.ds(r0, _CHUNK)], idx_v)
        cp_pe = pltpu.async_copy(pe_hbm.at[pl.ds(r0, _CHUNK)], pe_v, sem_pe)
        cp_gat = pltpu.async_copy(table_hbm.at[idx_v], gat_v, sem_gat)
        cp_pe.wait()
        cp_gat.wait()

        def add_body(k, carry):
            r = k // (_EMBED_DIM // _LANES)
            c = (k % (_EMBED_DIM // _LANES)) * _LANES
            sl = pl.ds(c, _LANES)
            gat_v[r, sl] = gat_v[r, sl] + pe_v[r, sl]
            return carry

        lax.fori_loop(0, _VECS_PER_CHUNK, add_body, 0)
        pltpu.sync_copy(gat_v, out_hbm.at[pl.ds(r0, _CHUNK)])


@jax.jit
def _encode(binary, embed_table, pe):
    mesh = plsc.VectorSubcoreMesh(core_axis_name="c", subcore_axis_name="s")
    f = functools.partial(
        pl.kernel,
        mesh=mesh,
        out_type=jax.ShapeDtypeStruct((_SEQ_LEN, _EMBED_DIM), jnp.float32),
        scratch_types=[
            pltpu.VMEM((_CHUNK,), jnp.int32),
            pltpu.VMEM((_CHUNK, _EMBED_DIM), jnp.float32),
            pltpu.VMEM((_CHUNK, _EMBED_DIM), jnp.float32),
            pltpu.SemaphoreType.DMA,
            pltpu.SemaphoreType.DMA,
        ],
    )(_sc_body)
    return f(binary, embed_table, pe)


def kernel(binary, embed_table):
    pe = jnp.asarray(_PE_SHIFTED)
    return _encode(binary, embed_table, pe)


# speedup vs baseline: 3.1506x; 3.1506x over previous
import functools

import jax
import jax.numpy as jnp
import numpy as np
from jax import lax
from jax.experimental import pallas as pl
from jax.experimental.pallas import tpu as pltpu
from jax.experimental.pallas import tpu_sc as plsc

_EMBED_DIM = 1024
_MAX_LENGTH = 4096
_SEQ_LEN = 4096

_NUM_CORES = 2
_NUM_SUBCORES = 16
_NUM_WORKERS = _NUM_CORES * _NUM_SUBCORES
_ROWS_PER_WORKER = _SEQ_LEN // _NUM_WORKERS
_CHUNK = 32
_STEPS = _ROWS_PER_WORKER // _CHUNK
_LANES = 16
_COLS = _EMBED_DIM // _LANES


def _pe_shifted_np() -> np.ndarray:
    d_model, max_len = _EMBED_DIM, _MAX_LENGTH
    position = np.arange(max_len, dtype=np.float64)[:, None]
    div_term = np.exp(
        np.arange(0, d_model, 2, dtype=np.float64) * (-np.log(10000.0) / d_model)
    )
    pe = np.zeros((max_len, d_model), dtype=np.float64)
    pe[:, 0::2] = np.sin(position * div_term)
    pe[:, 1::2] = np.cos(position * div_term)
    out = np.zeros((_SEQ_LEN, d_model), dtype=np.float64)
    out[1:] = pe[: _SEQ_LEN - 1]
    return out.astype(np.float32)


_PE_SHIFTED = _pe_shifted_np()


def _sc_body(bin_hbm, table_hbm, pe_hbm, out_hbm,
             bin_v, tab_v, buf0, buf1, sem_pe0, sem_pe1, sem_o0, sem_o1):
    wid = lax.axis_index("s") * _NUM_CORES + lax.axis_index("c")
    base = wid * _ROWS_PER_WORKER

    pltpu.sync_copy(bin_hbm.at[pl.ds(base, _ROWS_PER_WORKER)], bin_v)
    pltpu.sync_copy(table_hbm, tab_v)

    bufs = (buf0, buf1)
    pe_sems = (sem_pe0, sem_pe1)
    out_sems = (sem_o0, sem_o1)

    def step_compute(buf, t):
        for g in range(_CHUNK // 16):
            bv = bin_v[pl.ds(t * _CHUNK + g * 16, 16)].astype(jnp.float32)
            bfs = [jnp.full((16,), bv[r]) for r in range(16)]

            def col_body(c, carry, _bfs=bfs, _g=g):
                sl = pl.ds(c * _LANES, _LANES)
                e0 = tab_v[0, sl]
                d = tab_v[1, sl] - e0
                for r in range(16):
                    row = _g * 16 + r
                    buf[row, sl] = (buf[row, sl] + e0) + _bfs[r] * d
                return carry

            lax.fori_loop(0, _COLS, col_body, 0)

    cp_pe = [None] * _STEPS
    cp_out = [None] * _STEPS
    cp_pe[0] = pltpu.async_copy(
        pe_hbm.at[pl.ds(base, _CHUNK)], bufs[0], pe_sems[0])
    for t in range(_STEPS):
        p = t & 1
        if t + 1 < _STEPS:
            if t >= 1:
                cp_out[t - 1].wait()
            cp_pe[t + 1] = pltpu.async_copy(
                pe_hbm.at[pl.ds(base + (t + 1) * _CHUNK, _CHUNK)],
                bufs[1 - p], pe_sems[1 - p])
        cp_pe[t].wait()
        step_compute(bufs[p], t)
        cp_out[t] = pltpu.async_copy(
            bufs[p], out_hbm.at[pl.ds(base + t * _CHUNK, _CHUNK)], out_sems[p])
    cp_out[_STEPS - 2].wait()
    cp_out[_STEPS - 1].wait()


@jax.jit
def _encode(binary, embed_table, pe):
    mesh = plsc.VectorSubcoreMesh(core_axis_name="c", subcore_axis_name="s")
    f = functools.partial(
        pl.kernel,
        mesh=mesh,
        out_type=jax.ShapeDtypeStruct((_SEQ_LEN, _EMBED_DIM), jnp.float32),
        scratch_types=[
            pltpu.VMEM((_ROWS_PER_WORKER,), jnp.int32),
            pltpu.VMEM((2, _EMBED_DIM), jnp.float32),
            pltpu.VMEM((_CHUNK, _EMBED_DIM), jnp.float32),
            pltpu.VMEM((_CHUNK, _EMBED_DIM), jnp.float32),
            pltpu.SemaphoreType.DMA,
            pltpu.SemaphoreType.DMA,
            pltpu.SemaphoreType.DMA,
            pltpu.SemaphoreType.DMA,
        ],
    )(_sc_body)
    return f(binary, embed_table, pe)


def kernel(binary, embed_table):
    pe = jnp.asarray(_PE_SHIFTED)
    return _encode(binary, embed_table, pe)
